# baseline (speedup 1.0000x reference)
import functools

import jax
import jax.numpy as jnp
from jax import lax
from jax.experimental import pallas as pl
from jax.experimental.pallas import tpu as pltpu
from jax.experimental.pallas import tpu_sc as plsc

VOCAB_SIZE = 1000000
EMB_SIZE = 64
N = 16384

NUM_CORES = 2
NUM_SUBCORES = 16
NUM_WORKERS = NUM_CORES * NUM_SUBCORES
B_PER_W = N // NUM_WORKERS
IDX_CHUNK = 128
CHUNKS = B_PER_W // IDX_CHUNK


@functools.partial(
    pl.kernel,
    out_type=jax.ShapeDtypeStruct((N, 1, EMB_SIZE), jnp.float32),
    mesh=plsc.VectorSubcoreMesh(core_axis_name="c", subcore_axis_name="s"),
    scratch_types=[
        pltpu.VMEM((CHUNKS, IDX_CHUNK), jnp.int32),
        pltpu.VMEM((B_PER_W, EMB_SIZE), jnp.float32),
        pltpu.SemaphoreType.DMA,
        pltpu.SemaphoreType.DMA,
    ],
    compiler_params=pltpu.CompilerParams(use_tc_tiling_on_sc=False),
)
def _gather_kernel(ids_hbm, table_hbm, out_hbm, idx_v, rows_v, sem_g, sem_w):
    wid = lax.axis_index("s") * NUM_CORES + lax.axis_index("c")
    base = wid * B_PER_W
    pltpu.sync_copy(ids_hbm.at[pl.ds(wid * CHUNKS, CHUNKS)], idx_v)

    gathers = []
    for j in range(CHUNKS):
        gathers.append(
            pltpu.async_copy(
                table_hbm.at[idx_v.at[j]],
                rows_v.at[pl.ds(j * IDX_CHUNK, IDX_CHUNK)],
                sem_g,
            )
        )
    writes = []
    for j in range(CHUNKS):
        gathers[j].wait()
        writes.append(
            pltpu.async_copy(
                rows_v.at[pl.ds(j * IDX_CHUNK, IDX_CHUNK)],
                out_hbm.at[pl.ds(base + j * IDX_CHUNK, IDX_CHUNK), 0],
                sem_w,
            )
        )
    for w in writes:
        w.wait()


def kernel(vocab_ids, table):
    ids2d = vocab_ids.reshape(N // IDX_CHUNK, IDX_CHUNK)
    return _gather_kernel(ids2d, table)


# speedup vs baseline: 1.0700x; 1.0700x over previous
import functools

import jax
import jax.numpy as jnp
from jax import lax
from jax.experimental import pallas as pl
from jax.experimental.pallas import tpu as pltpu
from jax.experimental.pallas import tpu_sc as plsc

VOCAB_SIZE = 1000000
EMB_SIZE = 64
N = 16384

NUM_CORES = 2
NUM_SUBCORES = 16
NUM_WORKERS = NUM_CORES * NUM_SUBCORES
B_PER_W = N // NUM_WORKERS


@functools.partial(
    pl.kernel,
    out_type=jax.ShapeDtypeStruct((N, EMB_SIZE), jnp.float32),
    mesh=plsc.VectorSubcoreMesh(core_axis_name="c", subcore_axis_name="s"),
    scratch_types=[
        pltpu.VMEM((B_PER_W,), jnp.int32),
        pltpu.SemaphoreType.DMA,
    ],
)
def _gather_kernel(ids_hbm, table_hbm, out_hbm, idx_v, sem):
    wid = lax.axis_index("s") * NUM_CORES + lax.axis_index("c")
    base = wid * B_PER_W
    pltpu.sync_copy(ids_hbm.at[wid], idx_v)

    n_groups = B_PER_W // 16

    def body(g, carry):
        @pl.when(g < n_groups)
        def _issue():
            ids16 = idx_v[pl.ds(g * 16, 16)]
            for j in range(16):
                rid = ids16[j]
                pltpu.async_copy(
                    table_hbm.at[rid], out_hbm.at[base + g * 16 + j], sem
                )

        @pl.when(g > 0)
        def _drain():
            for j in range(16):
                pltpu.make_async_copy(
                    table_hbm.at[0], out_hbm.at[base], sem
                ).wait()

        return carry

    lax.fori_loop(0, n_groups + 1, body, 0)


def kernel(vocab_ids, table):
    ids2d = vocab_ids.reshape(NUM_WORKERS, B_PER_W)
    out = _gather_kernel(ids2d, table)
    return out.reshape(N, 1, EMB_SIZE)


# speedup vs baseline: 1.0705x; 1.0005x over previous
import functools

import jax
import jax.numpy as jnp
from jax import lax
from jax.experimental import pallas as pl
from jax.experimental.pallas import tpu as pltpu
from jax.experimental.pallas import tpu_sc as plsc

VOCAB_SIZE = 1000000
EMB_SIZE = 64
N = 16384

NUM_CORES = 2
NUM_SUBCORES = 16
NUM_WORKERS = NUM_CORES * NUM_SUBCORES
B_PER_W = N // NUM_WORKERS


@functools.partial(
    pl.kernel,
    out_type=jax.ShapeDtypeStruct((N, EMB_SIZE), jnp.float32),
    mesh=plsc.VectorSubcoreMesh(core_axis_name="c", subcore_axis_name="s"),
    scratch_types=[
        pltpu.VMEM((B_PER_W,), jnp.int32),
        pltpu.SemaphoreType.DMA,
    ],
)
def _gather_kernel(ids_hbm, table_hbm, out_hbm, idx_v, sem):
    wid = lax.axis_index("s") * NUM_CORES + lax.axis_index("c")
    base = wid * B_PER_W
    pltpu.sync_copy(ids_hbm.at[wid], idx_v)

    n_groups = B_PER_W // 16

    def body(g, carry):
        @pl.when(g < n_groups)
        def _issue():
            ids16 = idx_v[pl.ds(g * 16, 16)]
            for j in range(16):
                rid = ids16[j]
                pltpu.async_copy(
                    table_hbm.at[rid], out_hbm.at[base + g * 16 + j], sem
                )

        @pl.when(g > 0)
        def _drain():
            pltpu.make_async_copy(
                table_hbm.at[pl.ds(0, 16)], out_hbm.at[pl.ds(base, 16)], sem
            ).wait()

        return carry

    lax.fori_loop(0, n_groups + 1, body, 0)


def kernel(vocab_ids, table):
    ids2d = vocab_ids.reshape(NUM_WORKERS, B_PER_W)
    out = _gather_kernel(ids2d, table)
    return out.reshape(N, 1, EMB_SIZE)
